# baseline (speedup 1.0000x reference)
import functools

import jax
import jax.numpy as jnp
from jax import lax
from jax.experimental import pallas as pl
from jax.experimental.pallas import tpu as pltpu
from jax.experimental.pallas import tpu_sc as plsc

B = 16384
D = 32
L = 16
NC = 2
NS = 16
NW = NC * NS
BPW = B // NW
CH = 128
NCH = BPW // CH

_mesh = plsc.VectorSubcoreMesh(
    core_axis_name="c", subcore_axis_name="s", num_cores=NC, num_subcores=NS
)


@functools.partial(
    pl.kernel,
    out_type=jax.ShapeDtypeStruct((B,), jnp.float32),
    mesh=_mesh,
    compiler_params=pltpu.CompilerParams(
        needs_layout_passes=False, use_tc_tiling_on_sc=False
    ),
    scratch_types=[
        pltpu.VMEM((NCH, CH), jnp.int32),
        pltpu.VMEM((NCH, CH), jnp.int32),
        pltpu.VMEM((BPW, D), jnp.float32),
        pltpu.VMEM((BPW, D), jnp.float32),
        pltpu.VMEM((BPW,), jnp.float32),
        pltpu.VMEM((BPW,), jnp.float32),
        pltpu.VMEM((BPW,), jnp.float32),
        pltpu.SemaphoreType.DMA,
    ],
)
def _mf_sc(sess_hbm, aids_hbm, semb_hbm, aemb_hbm, sbias_hbm, abias_hbm,
           out_hbm, sidx_v, aidx_v, srows_v, arows_v, sb_v, ab_v, out_v, sem):
    wid = lax.axis_index("s") * NC + lax.axis_index("c")

    pltpu.sync_copy(sess_hbm.at[pl.ds(wid * NCH, NCH)], sidx_v)
    pltpu.sync_copy(aids_hbm.at[pl.ds(wid * NCH, NCH)], aidx_v)

    copies = []
    for j in range(NCH):
        dst = pl.ds(j * CH, CH)
        copies.append(pltpu.async_copy(semb_hbm.at[sidx_v.at[j]], srows_v.at[dst], sem))
        copies.append(pltpu.async_copy(aemb_hbm.at[aidx_v.at[j]], arows_v.at[dst], sem))
        copies.append(pltpu.async_copy(sbias_hbm.at[sidx_v.at[j]], sb_v.at[dst], sem))
        copies.append(pltpu.async_copy(abias_hbm.at[aidx_v.at[j]], ab_v.at[dst], sem))
    for c in copies:
        c.wait()

    iota = lax.iota(jnp.int32, L)

    @pl.loop(0, BPW // L)
    def _block(b):
        row0 = b * L
        rows = row0 + iota
        acc = sb_v[pl.ds(row0, L)] + ab_v[pl.ds(row0, L)]
        for d in range(D):
            col = jnp.full((L,), d, jnp.int32)
            acc = acc + (plsc.load_gather(srows_v, [rows, col])
                         * plsc.load_gather(arows_v, [rows, col]))
        out_v[pl.ds(row0, L)] = acc

    pltpu.sync_copy(out_v, out_hbm.at[pl.ds(wid * BPW, BPW)])


def kernel(sessions, aids, session_emb, aid_emb, session_bias, aid_bias):
    sess = sessions.reshape(B // CH, CH).astype(jnp.int32)
    aid = aids.reshape(B // CH, CH).astype(jnp.int32)
    sb = session_bias.reshape(-1)
    ab = aid_bias.reshape(-1)
    return _mf_sc(sess, aid, session_emb, aid_emb, sb, ab)
